# Initial kernel scaffold; baseline (speedup 1.0000x reference)
import jax
import jax.numpy as jnp
from jax.experimental import pallas as pl


def kernel(positional_encoding, t):
    raise NotImplementedError("write your pallas kernel here")


# speedup vs baseline: 1.9224x; 1.9224x over previous
import functools

import jax
import jax.numpy as jnp
from jax import lax
from jax.experimental import pallas as pl
from jax.experimental.pallas import tpu as pltpu
from jax.experimental.pallas import tpu_sc as plsc


def _make_gather(V, D, B):
    info = plsc.get_sparse_core_info()
    NC, NS = info.num_cores, info.num_subcores
    NW = NC * NS
    assert B % NW == 0
    b_per_w = B // NW
    C = 64
    n_chunks = b_per_w // C
    assert b_per_w % C == 0

    mesh = plsc.VectorSubcoreMesh(core_axis_name="c", subcore_axis_name="s")

    @functools.partial(
        pl.kernel,
        out_type=jax.ShapeDtypeStruct((B, D), jnp.float32),
        mesh=mesh,
        scratch_types=[
            pltpu.VMEM((b_per_w,), jnp.int32),
            pltpu.VMEM((C, D), jnp.float32),
            pltpu.SemaphoreType.DMA,
        ],
    )
    def gather_kernel(table_hbm, idx_hbm, out_hbm, idx_v, rows_v, sem):
        wid = lax.axis_index("s") * NC + lax.axis_index("c")
        base = wid * b_per_w
        pltpu.sync_copy(idx_hbm.at[pl.ds(base, b_per_w)], idx_v)
        for g in range(n_chunks):
            cp = pltpu.async_copy(
                table_hbm.at[idx_v.at[pl.ds(g * C, C)]], rows_v, sem
            )
            cp.wait()
            pltpu.sync_copy(rows_v, out_hbm.at[pl.ds(base + g * C, C)])

    return gather_kernel


def kernel(positional_encoding, t):
    V, D = positional_encoding.shape
    (B,) = t.shape
    gather = _make_gather(V, D, B)
    return gather(positional_encoding, t.astype(jnp.int32))


# speedup vs baseline: 1.9659x; 1.0226x over previous
import functools

import jax
import jax.numpy as jnp
from jax import lax
from jax.experimental import pallas as pl
from jax.experimental.pallas import tpu as pltpu
from jax.experimental.pallas import tpu_sc as plsc


def _make_gather(V, D, B):
    info = plsc.get_sparse_core_info()
    NC, NS = info.num_cores, info.num_subcores
    NW = NC * NS
    assert B % NW == 0
    b_per_w = B // NW
    C = 32
    n_chunks = b_per_w // C
    assert b_per_w % C == 0

    mesh = plsc.VectorSubcoreMesh(core_axis_name="c", subcore_axis_name="s")

    @functools.partial(
        pl.kernel,
        out_type=jax.ShapeDtypeStruct((B, D), jnp.float32),
        mesh=mesh,
        scratch_types=[
            pltpu.VMEM((b_per_w,), jnp.int32),
            pltpu.VMEM((C, D), jnp.float32),
            pltpu.VMEM((C, D), jnp.float32),
            pltpu.SemaphoreType.DMA,
        ],
    )
    def gather_kernel(table_hbm, idx_hbm, out_hbm, idx_v, rows_a, rows_b, sem):
        wid = lax.axis_index("s") * NC + lax.axis_index("c")
        base = wid * b_per_w
        pltpu.sync_copy(idx_hbm.at[pl.ds(base, b_per_w)], idx_v)
        bufs = (rows_a, rows_b)
        rd = [None, None]
        rd[0] = pltpu.async_copy(
            table_hbm.at[idx_v.at[pl.ds(0, C)]], bufs[0], sem
        )
        for g in range(n_chunks):
            b = g & 1
            rd[b].wait()
            if g + 1 < n_chunks:
                rd[1 - b] = pltpu.async_copy(
                    table_hbm.at[idx_v.at[pl.ds((g + 1) * C, C)]],
                    bufs[1 - b],
                    sem,
                )
            pltpu.sync_copy(bufs[b], out_hbm.at[pl.ds(base + g * C, C)])

    return gather_kernel


def kernel(positional_encoding, t):
    V, D = positional_encoding.shape
    (B,) = t.shape
    gather = _make_gather(V, D, B)
    return gather(positional_encoding, t.astype(jnp.int32))


# speedup vs baseline: 2.0314x; 1.0333x over previous
import functools

import jax
import jax.numpy as jnp
from jax import lax
from jax.experimental import pallas as pl
from jax.experimental.pallas import tpu as pltpu
from jax.experimental.pallas import tpu_sc as plsc


def _make_gather(V, D, B):
    info = plsc.get_sparse_core_info()
    NC, NS = info.num_cores, info.num_subcores
    NW = NC * NS
    assert B % NW == 0
    b_per_w = B // NW
    C = 32
    NB = 3
    n_chunks = b_per_w // C
    assert b_per_w % C == 0

    mesh = plsc.VectorSubcoreMesh(core_axis_name="c", subcore_axis_name="s")

    @functools.partial(
        pl.kernel,
        out_type=jax.ShapeDtypeStruct((B, D), jnp.float32),
        mesh=mesh,
        scratch_types=[
            pltpu.VMEM((b_per_w,), jnp.int32),
            *[pltpu.VMEM((C, D), jnp.float32) for _ in range(NB)],
            *[pltpu.SemaphoreType.DMA for _ in range(2 * NB)],
        ],
    )
    def gather_kernel(table_hbm, idx_hbm, out_hbm, idx_v, *bufs_and_sems):
        bufs = bufs_and_sems[:NB]
        gsem = bufs_and_sems[NB : 2 * NB]
        wsem = bufs_and_sems[2 * NB :]
        wid = lax.axis_index("s") * NC + lax.axis_index("c")
        base = wid * b_per_w
        pltpu.sync_copy(idx_hbm.at[pl.ds(base, b_per_w)], idx_v)

        def gather(g):
            b = g % NB
            return pltpu.async_copy(
                table_hbm.at[idx_v.at[pl.ds(g * C, C)]], bufs[b], gsem[b]
            )

        rd = {0: gather(0), 1: gather(1)}
        wr = {}
        for g in range(n_chunks):
            b = g % NB
            rd[g].wait()
            wr[g] = pltpu.async_copy(
                bufs[b], out_hbm.at[pl.ds(base + g * C, C)], wsem[b]
            )
            if g + NB - 1 < n_chunks:
                if g - 1 >= 0:
                    wr[g - 1].wait()
                rd[g + NB - 1] = gather(g + NB - 1)
        for g in range(n_chunks - NB + 1, n_chunks):
            wr[g - 1].wait()
        wr[n_chunks - 1].wait()

    return gather_kernel


def kernel(positional_encoding, t):
    V, D = positional_encoding.shape
    (B,) = t.shape
    gather = _make_gather(V, D, B)
    return gather(positional_encoding, t.astype(jnp.int32))
